# baseline (speedup 1.0000x reference)
import jax
import jax.numpy as jnp
from jax.experimental import pallas as pl
from jax.experimental.pallas import tpu as pltpu

D_MODEL = 3072
D_SPARSE = 24576
K_SPARSE = 128
BATCH = 128

_BN = 512
_NBLK = D_SPARSE // _BN
_BR = 32


def _select_rows(y, lane_max):
    rows = y.shape[0]
    k = jnp.int32(K_SPARSE)

    lo0 = jnp.min(lane_max, axis=1, keepdims=True)
    hi0 = jnp.max(lane_max, axis=1, keepdims=True) + 1
    cnt0 = jnp.sum((y >= lo0).astype(jnp.int32), axis=1, keepdims=True)

    def _active(lo, hi, cnt):
        d = jax.lax.bitcast_convert_type(hi - lo, jnp.uint32)
        return (cnt != k) & (d > jnp.uint32(1))

    def cond(state):
        lo, hi, cnt = state
        return jnp.any(_active(lo, hi, cnt))

    def body(state):
        lo, hi, cnt = state
        act = _active(lo, hi, cnt)
        mid = (lo & hi) + ((lo ^ hi) >> 1)
        c = jnp.sum((y >= mid).astype(jnp.int32), axis=1, keepdims=True)
        ge = c >= k
        lo = jnp.where(act & ge, mid, lo)
        cnt = jnp.where(act & ge, c, cnt)
        hi = jnp.where(act & (~ge), mid, hi)
        return lo, hi, cnt

    t, _, cnt_ge = jax.lax.while_loop(cond, body, (lo0, hi0, cnt0))

    m_out = jnp.full((rows, 1), jnp.int32(D_SPARSE - 1))

    def _tie_m(_):
        cnt_gt = jnp.sum((y > t).astype(jnp.int32), axis=1, keepdims=True)
        extras = jnp.int32(K_SPARSE) - cnt_gt
        idx = jax.lax.broadcasted_iota(jnp.int32, y.shape, 1)
        tie = y == t

        def ibody(i, m):
            b = 14 - i
            c = m + (jnp.int32(1) << b)
            cnt = jnp.sum((tie & (idx <= c)).astype(jnp.int32), axis=1,
                          keepdims=True)
            return jnp.where(cnt < extras, c, m)

        m = jax.lax.fori_loop(0, 15, ibody,
                              jnp.full((rows, 1), jnp.int32(-1)))
        return jnp.where(cnt_ge == k, jnp.int32(D_SPARSE - 1), m + 1)

    m_out = jax.lax.cond(jnp.all(cnt_ge == k), lambda _: m_out, _tie_m,
                         operand=None)
    return t, m_out


def _fused_kernel(h_ref, w_ref, pb_ref, eb_ref, out_ref, lmax_ref):
    i = pl.program_id(0)

    x = h_ref[...] - pb_ref[...]
    acts = jax.lax.dot_general(
        x, w_ref[...],
        dimension_numbers=(((1,), (1,)), ((), ())),
        preferred_element_type=jnp.float32,
    ) + eb_ref[...]
    s = jax.lax.bitcast_convert_type(acts, jnp.int32)
    y = jnp.where(s >= 0, s, s ^ jnp.int32(0x7FFFFFFF))
    out_ref[:, pl.ds(i * _BN, _BN)] = jax.lax.bitcast_convert_type(
        y, jnp.float32)

    lm = jnp.max(y.reshape(BATCH, _BN // 128, 128), axis=1)

    @pl.when(i == 0)
    def _():
        lmax_ref[...] = lm

    @pl.when(i > 0)
    def _():
        lmax_ref[...] = jnp.maximum(lmax_ref[...], lm)

    @pl.when(i == _NBLK - 1)
    def _():
        def rbody(r, _):
            yb = jax.lax.bitcast_convert_type(
                out_ref[pl.ds(r * _BR, _BR), :], jnp.int32)
            lmb = lmax_ref[pl.ds(r * _BR, _BR), :]
            t, m = _select_rows(yb, lmb)
            idx = jax.lax.broadcasted_iota(jnp.int32, yb.shape, 1)
            keep = ((yb > t) | ((yb == t) & (idx <= m))) & (yb > 0)
            out_ref[pl.ds(r * _BR, _BR), :] = jnp.where(
                keep, jax.lax.bitcast_convert_type(yb, jnp.float32), 0.0)
            return 0

        jax.lax.fori_loop(0, BATCH // _BR, rbody, 0)


@jax.jit
def kernel(h, W_enc, pre_bias, enc_bias):
    pb = pre_bias.reshape(1, D_MODEL)
    eb = enc_bias.reshape(1, D_SPARSE)

    return pl.pallas_call(
        _fused_kernel,
        grid=(_NBLK,),
        in_specs=[
            pl.BlockSpec((BATCH, D_MODEL), lambda i: (0, 0)),
            pl.BlockSpec((_BN, D_MODEL), lambda i: (i, 0)),
            pl.BlockSpec((1, D_MODEL), lambda i: (0, 0)),
            pl.BlockSpec((1, _BN), lambda i: (0, i)),
        ],
        out_specs=pl.BlockSpec((BATCH, D_SPARSE), lambda i: (0, 0)),
        out_shape=jax.ShapeDtypeStruct((BATCH, D_SPARSE), jnp.float32),
        scratch_shapes=[pltpu.VMEM((BATCH, 128), jnp.int32)],
    )(h, W_enc, pb, eb)


# speedup vs baseline: 1.0875x; 1.0875x over previous
import jax
import jax.numpy as jnp
from jax.experimental import pallas as pl

D_MODEL = 3072
D_SPARSE = 24576
K_SPARSE = 128
BATCH = 128

_BN = 1024
_BR = 32


def _matmul_kernel(h_ref, w_ref, pb_ref, eb_ref, out_ref, lmax_ref):
    i = pl.program_id(0)
    x = h_ref[...] - pb_ref[...]
    acts = jax.lax.dot_general(
        x, w_ref[...],
        dimension_numbers=(((1,), (1,)), ((), ())),
        preferred_element_type=jnp.float32,
    ) + eb_ref[...]
    s = jax.lax.bitcast_convert_type(acts, jnp.int32)
    y = jnp.where(s >= 0, s, s ^ jnp.int32(0x7FFFFFFF))
    out_ref[...] = y

    lm = jnp.max(y.reshape(BATCH, _BN // 128, 128), axis=1)

    @pl.when(i == 0)
    def _():
        lmax_ref[...] = lm

    @pl.when(i > 0)
    def _():
        lmax_ref[...] = jnp.maximum(lmax_ref[...], lm)


def _select_kernel(y_ref, lmax_ref, out_ref):
    y = y_ref[...]
    rows = y.shape[0]
    k = jnp.int32(K_SPARSE)

    lane_max = lmax_ref[...]
    lo0 = jnp.min(lane_max, axis=1, keepdims=True)
    hi0 = jnp.max(lane_max, axis=1, keepdims=True) + 1
    cnt0 = jnp.sum((y >= lo0).astype(jnp.int32), axis=1, keepdims=True)

    def _active(lo, hi, cnt):
        d = jax.lax.bitcast_convert_type(hi - lo, jnp.uint32)
        return (cnt != k) & (d > jnp.uint32(1))

    def cond(state):
        lo, hi, cnt, _nhi, _it = state
        return jnp.any(_active(lo, hi, cnt))

    def body(state):
        lo, hi, cnt, nhi, it = state
        act = _active(lo, hi, cnt)
        width = (hi - lo).astype(jnp.float32)
        frac = (cnt - k).astype(jnp.float32) / (cnt - nhi).astype(jnp.float32)
        delta = (frac * width).astype(jnp.int32)
        mid_rf = lo + jnp.clip(delta, 1, hi - lo - 1)
        mid_bi = (lo & hi) + ((lo ^ hi) >> 1)
        mid = jnp.where(it % 3 == 2, mid_bi, mid_rf)
        c = jnp.sum((y >= mid).astype(jnp.int32), axis=1, keepdims=True)
        ge = c >= k
        lo = jnp.where(act & ge, mid, lo)
        cnt = jnp.where(act & ge, c, cnt)
        hi = jnp.where(act & (~ge), mid, hi)
        nhi = jnp.where(act & (~ge), c, nhi)
        return lo, hi, cnt, nhi, it + 1

    nhi0 = jnp.zeros((rows, 1), jnp.int32)
    t, _, cnt_ge, _, _ = jax.lax.while_loop(
        cond, body, (lo0, hi0, cnt0, nhi0, jnp.int32(0)))

    out_ref[...] = jnp.where(
        (y >= t) & (y > 0), jax.lax.bitcast_convert_type(y, jnp.float32),
        0.0)

    @pl.when(jnp.logical_not(jnp.all(cnt_ge == k)))
    def _():
        cnt_gt = jnp.sum((y > t).astype(jnp.int32), axis=1, keepdims=True)
        extras = k - cnt_gt
        idx = jax.lax.broadcasted_iota(jnp.int32, y.shape, 1)
        tie = y == t

        def ibody(i, m):
            b = 14 - i
            c = m + (jnp.int32(1) << b)
            cnt = jnp.sum((tie & (idx <= c)).astype(jnp.int32), axis=1,
                          keepdims=True)
            return jnp.where(cnt < extras, c, m)

        m0 = jnp.full((rows, 1), jnp.int32(-1))
        m = jax.lax.fori_loop(0, 15, ibody, m0)

        keep = ((y > t) | (tie & (idx <= m + 1))) & (y > 0)
        out_ref[...] = jnp.where(
            keep, jax.lax.bitcast_convert_type(y, jnp.float32), 0.0)


@jax.jit
def kernel(h, W_enc, pre_bias, enc_bias):
    pb = pre_bias.reshape(1, D_MODEL)
    eb = enc_bias.reshape(1, D_SPARSE)

    keys, lmax = pl.pallas_call(
        _matmul_kernel,
        grid=(D_SPARSE // _BN,),
        in_specs=[
            pl.BlockSpec((BATCH, D_MODEL), lambda i: (0, 0)),
            pl.BlockSpec((_BN, D_MODEL), lambda i: (i, 0)),
            pl.BlockSpec((1, D_MODEL), lambda i: (0, 0)),
            pl.BlockSpec((1, _BN), lambda i: (0, i)),
        ],
        out_specs=[pl.BlockSpec((BATCH, _BN), lambda i: (0, i)),
                   pl.BlockSpec((BATCH, 128), lambda i: (0, 0))],
        out_shape=[jax.ShapeDtypeStruct((BATCH, D_SPARSE), jnp.int32),
                   jax.ShapeDtypeStruct((BATCH, 128), jnp.int32)],
    )(h, W_enc, pb, eb)

    out = pl.pallas_call(
        _select_kernel,
        grid=(BATCH // _BR,),
        in_specs=[pl.BlockSpec((_BR, D_SPARSE), lambda i: (i, 0)),
                  pl.BlockSpec((_BR, 128), lambda i: (i, 0))],
        out_specs=pl.BlockSpec((_BR, D_SPARSE), lambda i: (i, 0)),
        out_shape=jax.ShapeDtypeStruct((BATCH, D_SPARSE), jnp.float32),
    )(keys, lmax)
    return out
